# baseline (speedup 1.0000x reference)
import functools

import jax
import jax.numpy as jnp
from jax import lax
from jax.experimental import pallas as pl
from jax.experimental.pallas import tpu as pltpu
from jax.experimental.pallas import tpu_sc as plsc

F = 26
V = 100000
E = 16
B = 16384
BF = B * F

NC = 2
NS = 16
NW = NC * NS

CV = 2048
NJ = V // CV
VT = NJ * CV
TAILV = V - VT
NE_ITEMS = (F // 2) * NJ
NO_ITEMS = (F // 2) * NJ
ROWS_PER_CHUNK = CV // 8

RPW = B // NW
PER_W = BF // NW
G = 128
GRP = 8
NGRP = PER_W // (GRP * G)


def _sc_relayout(tablesT, tails_lin):
    mesh = plsc.VectorSubcoreMesh(core_axis_name="c", subcore_axis_name="s")

    @functools.partial(
        pl.kernel,
        out_type=jax.ShapeDtypeStruct((F * V * E // 128, 128), jnp.float32),
        mesh=mesh,
        scratch_types=[
            pltpu.VMEM((E, CV + 128), jnp.float32),
            pltpu.VMEM((E, 128), jnp.float32),
            pltpu.VMEM((8, 128), jnp.float32),
            pltpu.VMEM((ROWS_PER_CHUNK, 128), jnp.float32),
            pltpu.SemaphoreType.DMA,
        ],
        compiler_params=pltpu.CompilerParams(
            use_tc_tiling_on_sc=True, needs_layout_passes=False
        ),
    )
    def k(tab_hbm, tails_hbm, out_hbm, in_a, in_b, tails_v, out_a, sem):
        wid = lax.axis_index("s") * NC + lax.axis_index("c")
        iota = lax.iota(jnp.int32, 16)

        def transpose_rows(nrows, colbase):
            @plsc.parallel_loop(0, nrows // 2, unroll=8)
            def bbody(b):
                c0 = colbase + b * 16
                r0 = 2 * b
                for d in range(16):
                    idxv = jnp.bitwise_and(iota + d, 15)
                    colv = c0 + idxv
                    rowv = r0 + lax.shift_right_logical(idxv, 3)
                    col2 = lax.shift_left(jnp.bitwise_and(idxv, 7), 4) + iota
                    g = plsc.load_gather(in_a, [iota, colv])
                    plsc.store_scatter(out_a, [rowv, col2], g)

        def ebody(t, carry):
            i = wid + NW * t

            @pl.when(i < NE_ITEMS)
            def _():
                fq = lax.shift_right_logical(i * 683, 15)
                j = i - fq * NJ
                f = 2 * fq
                v0 = pl.multiple_of(j * CV, 128)
                pltpu.sync_copy(
                    tab_hbm.at[f, :, pl.ds(v0, CV)],
                    in_a.at[:, pl.ds(0, CV)],
                )
                transpose_rows(ROWS_PER_CHUNK, 0)
                row0 = pl.multiple_of(
                    f * (V // 8) + j * ROWS_PER_CHUNK, 8
                )
                pltpu.sync_copy(out_a, out_hbm.at[pl.ds(row0, ROWS_PER_CHUNK)])

            return carry

        lax.fori_loop(0, (NE_ITEMS + NW - 1) // NW, ebody, 0)

        def obody(t, carry):
            i = wid + NW * t

            @pl.when(i < NO_ITEMS)
            def _():
                fq = lax.shift_right_logical(i * 683, 15)
                j = i - fq * NJ
                f = 2 * fq + 1
                v0 = pl.multiple_of(j * CV, 128)
                pltpu.sync_copy(
                    tab_hbm.at[f, :, pl.ds(v0, CV + 128)],
                    in_a,
                )
                transpose_rows(ROWS_PER_CHUNK, 32)
                row0 = pl.multiple_of(
                    f * (V // 8) + 4 + j * ROWS_PER_CHUNK, 8
                )
                pltpu.sync_copy(out_a, out_hbm.at[pl.ds(row0, ROWS_PER_CHUNK)])

            return carry

        lax.fori_loop(0, (NO_ITEMS + NW - 1) // NW, obody, 0)

        @pl.when(wid < F // 2)
        def _():
            f = 2 * wid + 1
            pltpu.sync_copy(
                tab_hbm.at[f, :, pl.ds(VT, TAILV - 32)],
                in_a.at[:, pl.ds(0, TAILV - 32)],
            )
            pltpu.sync_copy(tails_hbm.at[pl.ds(8 * wid, 8)], tails_v)
            n1 = (TAILV - 64) // 8
            transpose_rows(n1, 32)
            for r in range(4):
                for l in range(8):
                    out_a[n1 + r, pl.ds(l * 16, 16)] = tails_v[
                        4 + r, pl.ds(l * 16, 16)
                    ]
            row0 = pl.multiple_of(f * (V // 8) + 4 + NJ * ROWS_PER_CHUNK, 8)
            pltpu.sync_copy(
                out_a.at[pl.ds(0, n1 + 4)], out_hbm.at[pl.ds(row0, n1 + 4)]
            )

        @pl.when(jnp.logical_and(wid >= F // 2, wid < F))
        def _():
            kk = wid - F // 2
            f0 = 2 * kk
            pltpu.sync_copy(
                tab_hbm.at[f0, :, pl.ds(VT, TAILV - 32)],
                in_a.at[:, pl.ds(0, TAILV - 32)],
            )
            pltpu.sync_copy(tails_hbm.at[pl.ds(8 * kk, 8)], tails_v)
            pltpu.sync_copy(tab_hbm.at[f0 + 1, :, pl.ds(0, 128)], in_b)
            n1 = (TAILV - 32) // 8
            transpose_rows(n1, 0)
            for r in range(4):
                for l in range(8):
                    out_a[n1 + r, pl.ds(l * 16, 16)] = tails_v[
                        r, pl.ds(l * 16, 16)
                    ]
            for r in range(4):
                for jj in range(8):
                    col = jnp.broadcast_to(r * 8 + jj, (16,))
                    g = plsc.load_gather(in_b, [iota, col])
                    out_a[n1 + 4 + r, pl.ds(jj * 16, 16)] = g
            row0 = pl.multiple_of(f0 * (V // 8) + VT // 8, 8)
            pltpu.sync_copy(
                out_a.at[pl.ds(0, n1 + 8)], out_hbm.at[pl.ds(row0, n1 + 8)]
            )

    return k(tablesT, tails_lin)


def _sc_gather(flat_tables, ids):
    mesh = plsc.VectorSubcoreMesh(core_axis_name="c", subcore_axis_name="s")

    @functools.partial(
        pl.kernel,
        out_type=jax.ShapeDtypeStruct((BF, E), jnp.float32),
        mesh=mesh,
        scratch_types=[
            pltpu.VMEM((RPW, F), jnp.int32),
            pltpu.VMEM((PER_W // G, G), jnp.int32),
            pltpu.VMEM((GRP * G, E), jnp.float32),
            pltpu.SemaphoreType.DMA,
            pltpu.SemaphoreType.DMA,
        ],
        compiler_params=pltpu.CompilerParams(
            use_tc_tiling_on_sc=False, needs_layout_passes=False
        ),
    )
    def k(tab_hbm, ids_hbm, out_hbm, ids_v, idx_v, buf, gsem, osem):
        wid = lax.axis_index("s") * NC + lax.axis_index("c")
        rbase = wid * RPW
        base = wid * PER_W
        pltpu.sync_copy(ids_hbm.at[pl.ds(rbase, RPW)], ids_v)

        iota = lax.iota(jnp.int32, 16)

        def cbody(g, carry):
            for l in range(G // 16):
                p = g * G + l * 16 + iota
                r = lax.shift_right_logical(p * 20165, 19)
                col = p - r * F
                v = plsc.load_gather(ids_v, [r, col])
                idx_v[g, pl.ds(l * 16, 16)] = v + col * V
            return carry

        lax.fori_loop(0, PER_W // G, cbody, 0)

        def gbody(gp, carry):
            handles = []
            for j in range(GRP):
                h = pltpu.async_copy(
                    tab_hbm.at[idx_v.at[gp * GRP + j]],
                    buf.at[pl.ds(j * G, G)],
                    gsem,
                )
                handles.append(h)
            for h in handles:
                h.wait()
            out = pltpu.async_copy(
                buf, out_hbm.at[pl.ds(base + gp * (GRP * G), GRP * G)], osem
            )
            out.wait()
            return carry

        lax.fori_loop(0, NGRP, gbody, 0)

    return k(flat_tables, ids)


def _tc_mlp(x, W1, b1, W2, b2, W3, b3):
    BLK = 1024
    grid = B // BLK

    def body(x_ref, w1_ref, b1_ref, w2_ref, b2_ref, w3_ref, b3_ref, o_ref):
        xb = x_ref[...]
        h = jnp.dot(xb, w1_ref[...], preferred_element_type=jnp.float32)
        h = jnp.maximum(h + b1_ref[...], 0.0)
        h = jnp.dot(h, w2_ref[...], preferred_element_type=jnp.float32)
        h = jnp.maximum(h + b2_ref[...], 0.0)
        o = jnp.dot(h, w3_ref[...], preferred_element_type=jnp.float32)
        o_ref[...] = jax.nn.sigmoid(o + b3_ref[...])

    out = pl.pallas_call(
        body,
        grid=(grid,),
        in_specs=[
            pl.BlockSpec((BLK, F * E), lambda i: (i, 0)),
            pl.BlockSpec((F * E, 256), lambda i: (0, 0)),
            pl.BlockSpec((1, 256), lambda i: (0, 0)),
            pl.BlockSpec((256, 128), lambda i: (0, 0)),
            pl.BlockSpec((1, 128), lambda i: (0, 0)),
            pl.BlockSpec((128, 1), lambda i: (0, 0)),
            pl.BlockSpec((1, 1), lambda i: (0, 0)),
        ],
        out_specs=pl.BlockSpec((BLK, 1), lambda i: (i, 0)),
        out_shape=jax.ShapeDtypeStruct((B, 1), jnp.float32),
    )(x, W1, b1.reshape(1, 256), W2, b2.reshape(1, 128), W3, b3.reshape(1, 1))
    return out[:, 0]


def kernel(ids, tables, W1, b1, W2, b2, W3, b3):
    tablesT = tables.transpose(0, 2, 1)
    tails_lin = tables[:, V - 32:, :].reshape(F * 32 * E // 128, 128)
    tabL = _sc_relayout(tablesT, tails_lin)
    flat = tabL.reshape(F * V, E)
    rows = _sc_gather(flat, ids.astype(jnp.int32))
    x = rows.reshape(B, F * E)
    return _tc_mlp(x, W1, b1, W2, b2, W3, b3)


# speedup vs baseline: 1.0777x; 1.0777x over previous
import functools

import jax
import jax.numpy as jnp
from jax import lax
from jax.experimental import pallas as pl
from jax.experimental.pallas import tpu as pltpu
from jax.experimental.pallas import tpu_sc as plsc

F = 26
V = 100000
E = 16
B = 16384
BF = B * F

NC = 2
NS = 16
NW = NC * NS

CV = 2048
NJ = V // CV
VT = NJ * CV
TAILV = V - VT
NE_ITEMS = (F // 2) * NJ
NO_ITEMS = (F // 2) * NJ
ROWS_PER_CHUNK = CV // 8

RPW = B // NW
PER_W = BF // NW
G = 128
GRP = 8
NGRP = PER_W // (GRP * G)


def _sc_relayout(tablesT, tails_lin):
    mesh = plsc.VectorSubcoreMesh(core_axis_name="c", subcore_axis_name="s")

    @functools.partial(
        pl.kernel,
        out_type=jax.ShapeDtypeStruct((F * V * E // 128, 128), jnp.float32),
        mesh=mesh,
        scratch_types=[
            pltpu.VMEM((E, CV + 128), jnp.float32),
            pltpu.VMEM((E, 128), jnp.float32),
            pltpu.VMEM((8, 128), jnp.float32),
            pltpu.VMEM((ROWS_PER_CHUNK, 128), jnp.float32),
            pltpu.SemaphoreType.DMA,
        ],
        compiler_params=pltpu.CompilerParams(
            use_tc_tiling_on_sc=True, needs_layout_passes=False
        ),
    )
    def k(tab_hbm, tails_hbm, out_hbm, in_a, in_b, tails_v, out_a, sem):
        wid = lax.axis_index("s") * NC + lax.axis_index("c")
        iota = lax.iota(jnp.int32, 16)

        def transpose_rows(nrows, colbase):
            @plsc.parallel_loop(0, nrows // 2, unroll=4)
            def bbody(b):
                c0 = colbase + b * 16
                r0 = 2 * b
                for d in range(16):
                    idxv = jnp.bitwise_and(iota + d, 15)
                    colv = c0 + idxv
                    rowv = r0 + lax.shift_right_logical(idxv, 3)
                    col2 = lax.shift_left(jnp.bitwise_and(idxv, 7), 4) + iota
                    g = plsc.load_gather(in_a, [iota, colv])
                    plsc.store_scatter(out_a, [rowv, col2], g)

        def ebody(t, carry):
            i = wid + NW * t

            @pl.when(i < NE_ITEMS)
            def _():
                fq = lax.shift_right_logical(i * 683, 15)
                j = i - fq * NJ
                f = 2 * fq
                v0 = pl.multiple_of(j * CV, 128)
                pltpu.sync_copy(
                    tab_hbm.at[f, :, pl.ds(v0, CV)],
                    in_a.at[:, pl.ds(0, CV)],
                )
                transpose_rows(ROWS_PER_CHUNK, 0)
                row0 = pl.multiple_of(
                    f * (V // 8) + j * ROWS_PER_CHUNK, 8
                )
                pltpu.sync_copy(out_a, out_hbm.at[pl.ds(row0, ROWS_PER_CHUNK)])

            return carry

        lax.fori_loop(0, (NE_ITEMS + NW - 1) // NW, ebody, 0)

        def obody(t, carry):
            i = wid + NW * t

            @pl.when(i < NO_ITEMS)
            def _():
                fq = lax.shift_right_logical(i * 683, 15)
                j = i - fq * NJ
                f = 2 * fq + 1
                v0 = pl.multiple_of(j * CV, 128)
                pltpu.sync_copy(
                    tab_hbm.at[f, :, pl.ds(v0, CV + 128)],
                    in_a,
                )
                transpose_rows(ROWS_PER_CHUNK, 32)
                row0 = pl.multiple_of(
                    f * (V // 8) + 4 + j * ROWS_PER_CHUNK, 8
                )
                pltpu.sync_copy(out_a, out_hbm.at[pl.ds(row0, ROWS_PER_CHUNK)])

            return carry

        lax.fori_loop(0, (NO_ITEMS + NW - 1) // NW, obody, 0)

        @pl.when(wid < F // 2)
        def _():
            f = 2 * wid + 1
            pltpu.sync_copy(
                tab_hbm.at[f, :, pl.ds(VT, TAILV - 32)],
                in_a.at[:, pl.ds(0, TAILV - 32)],
            )
            pltpu.sync_copy(tails_hbm.at[pl.ds(8 * wid, 8)], tails_v)
            n1 = (TAILV - 64) // 8
            transpose_rows(n1, 32)
            for r in range(4):
                for l in range(8):
                    out_a[n1 + r, pl.ds(l * 16, 16)] = tails_v[
                        4 + r, pl.ds(l * 16, 16)
                    ]
            row0 = pl.multiple_of(f * (V // 8) + 4 + NJ * ROWS_PER_CHUNK, 8)
            pltpu.sync_copy(
                out_a.at[pl.ds(0, n1 + 4)], out_hbm.at[pl.ds(row0, n1 + 4)]
            )

        @pl.when(jnp.logical_and(wid >= F // 2, wid < F))
        def _():
            kk = wid - F // 2
            f0 = 2 * kk
            pltpu.sync_copy(
                tab_hbm.at[f0, :, pl.ds(VT, TAILV - 32)],
                in_a.at[:, pl.ds(0, TAILV - 32)],
            )
            pltpu.sync_copy(tails_hbm.at[pl.ds(8 * kk, 8)], tails_v)
            pltpu.sync_copy(tab_hbm.at[f0 + 1, :, pl.ds(0, 128)], in_b)
            n1 = (TAILV - 32) // 8
            transpose_rows(n1, 0)
            for r in range(4):
                for l in range(8):
                    out_a[n1 + r, pl.ds(l * 16, 16)] = tails_v[
                        r, pl.ds(l * 16, 16)
                    ]
            for r in range(4):
                for jj in range(8):
                    col = jnp.broadcast_to(r * 8 + jj, (16,))
                    g = plsc.load_gather(in_b, [iota, col])
                    out_a[n1 + 4 + r, pl.ds(jj * 16, 16)] = g
            row0 = pl.multiple_of(f0 * (V // 8) + VT // 8, 8)
            pltpu.sync_copy(
                out_a.at[pl.ds(0, n1 + 8)], out_hbm.at[pl.ds(row0, n1 + 8)]
            )

    return k(tablesT, tails_lin)


def _sc_gather(flat_tables, ids):
    mesh = plsc.VectorSubcoreMesh(core_axis_name="c", subcore_axis_name="s")

    @functools.partial(
        pl.kernel,
        out_type=jax.ShapeDtypeStruct((BF, E), jnp.float32),
        mesh=mesh,
        scratch_types=[
            pltpu.VMEM((RPW, F), jnp.int32),
            pltpu.VMEM((PER_W // G, G), jnp.int32),
            pltpu.VMEM((GRP * G, E), jnp.float32),
            pltpu.SemaphoreType.DMA,
            pltpu.SemaphoreType.DMA,
        ],
        compiler_params=pltpu.CompilerParams(
            use_tc_tiling_on_sc=False, needs_layout_passes=False
        ),
    )
    def k(tab_hbm, ids_hbm, out_hbm, ids_v, idx_v, buf, gsem, osem):
        wid = lax.axis_index("s") * NC + lax.axis_index("c")
        rbase = wid * RPW
        base = wid * PER_W
        pltpu.sync_copy(ids_hbm.at[pl.ds(rbase, RPW)], ids_v)

        iota = lax.iota(jnp.int32, 16)

        @plsc.parallel_loop(0, PER_W // G, unroll=2)
        def cbody(g):
            for l in range(G // 16):
                p = g * G + l * 16 + iota
                r = lax.shift_right_logical(p * 20165, 19)
                col = p - r * F
                v = plsc.load_gather(ids_v, [r, col])
                idx_v[g, pl.ds(l * 16, 16)] = v + col * V

        def gbody(gp, carry):
            handles = []
            for j in range(GRP):
                h = pltpu.async_copy(
                    tab_hbm.at[idx_v.at[gp * GRP + j]],
                    buf.at[pl.ds(j * G, G)],
                    gsem,
                )
                handles.append(h)
            for h in handles:
                h.wait()
            out = pltpu.async_copy(
                buf, out_hbm.at[pl.ds(base + gp * (GRP * G), GRP * G)], osem
            )
            out.wait()
            return carry

        lax.fori_loop(0, NGRP, gbody, 0)

    return k(flat_tables, ids)


def _tc_mlp(x, W1, b1, W2, b2, W3, b3):
    BLK = 1024
    grid = B // BLK

    def body(x_ref, w1_ref, b1_ref, w2_ref, b2_ref, w3_ref, b3_ref, o_ref):
        xb = x_ref[...]
        h = jnp.dot(xb, w1_ref[...], preferred_element_type=jnp.float32)
        h = jnp.maximum(h + b1_ref[...], 0.0)
        h = jnp.dot(h, w2_ref[...], preferred_element_type=jnp.float32)
        h = jnp.maximum(h + b2_ref[...], 0.0)
        o = jnp.dot(h, w3_ref[...], preferred_element_type=jnp.float32)
        o_ref[...] = jax.nn.sigmoid(o + b3_ref[...])

    out = pl.pallas_call(
        body,
        grid=(grid,),
        in_specs=[
            pl.BlockSpec((BLK, F * E), lambda i: (i, 0)),
            pl.BlockSpec((F * E, 256), lambda i: (0, 0)),
            pl.BlockSpec((1, 256), lambda i: (0, 0)),
            pl.BlockSpec((256, 128), lambda i: (0, 0)),
            pl.BlockSpec((1, 128), lambda i: (0, 0)),
            pl.BlockSpec((128, 1), lambda i: (0, 0)),
            pl.BlockSpec((1, 1), lambda i: (0, 0)),
        ],
        out_specs=pl.BlockSpec((BLK, 1), lambda i: (i, 0)),
        out_shape=jax.ShapeDtypeStruct((B, 1), jnp.float32),
    )(x, W1, b1.reshape(1, 256), W2, b2.reshape(1, 128), W3, b3.reshape(1, 1))
    return out[:, 0]


def kernel(ids, tables, W1, b1, W2, b2, W3, b3):
    tablesT = tables.transpose(0, 2, 1)
    tails_lin = tables[:, V - 32:, :].reshape(F * 32 * E // 128, 128)
    tabL = _sc_relayout(tablesT, tails_lin)
    flat = tabL.reshape(F * V, E)
    rows = _sc_gather(flat, ids.astype(jnp.int32))
    x = rows.reshape(B, F * E)
    return _tc_mlp(x, W1, b1, W2, b2, W3, b3)
